# Initial kernel scaffold; baseline (speedup 1.0000x reference)
import jax
import jax.numpy as jnp
from jax.experimental import pallas as pl


def kernel(node_embed, edges, W, b):
    raise NotImplementedError("write your pallas kernel here")


# speedup vs baseline: 13.5446x; 13.5446x over previous
import functools

import jax
import jax.numpy as jnp
from jax import lax
from jax.experimental import pallas as pl
from jax.experimental.pallas import tpu as pltpu
from jax.experimental.pallas import tpu_sc as plsc

N = 10000
E = 320000
D = 128

NC = 2
NS = 16
NW = NC * NS
K = 128
G = -(-E // (NW * K))
E_PAD = NW * G * K
NB = 79
NPAD = NB * K
PAD_ROW = 10048
RPT = NPAD // NS

_mesh = plsc.VectorSubcoreMesh(core_axis_name="c", subcore_axis_name="s")


@functools.partial(
    pl.kernel,
    out_type=jax.ShapeDtypeStruct((NC, NB, K), jnp.float32),
    mesh=_mesh,
    compiler_params=pltpu.CompilerParams(needs_layout_passes=False),
    scratch_types=[
        pltpu.VMEM((G, K), jnp.int32),
        pltpu.VMEM((NB, K), jnp.float32),
        pltpu.VMEM((NB,), jnp.int32),
        pltpu.VMEM_SHARED((NB, K), jnp.float32),
    ],
)
def _deg_kernel(dst_hbm, iota_hbm, zeros_hbm, deg_out, dst_v, deg_v, idt_v, acc):
    c = lax.axis_index("c")
    s = lax.axis_index("s")
    t = c * NS + s

    @pl.when(s == 0)
    def _():
        pltpu.sync_copy(zeros_hbm, acc)

    pltpu.sync_copy(dst_hbm.at[t], dst_v)
    pltpu.sync_copy(iota_hbm, idt_v)

    @pl.loop(0, NB)
    def _(r):
        @pl.loop(0, K // 16)
        def _(j):
            deg_v[r, pl.ds(j * 16, 16)] = jnp.zeros((16,), jnp.float32)

    ones = jnp.ones((16,), jnp.float32)

    @pl.loop(0, G)
    def _(g):
        @pl.loop(0, K // 16)
        def _(j):
            idx = dst_v[g, pl.ds(j * 16, 16)]
            plsc.addupdate_scatter(deg_v, [idx >> 7, idx & 127], ones)

    plsc.subcore_barrier()
    pltpu.sync_copy(deg_v, acc.at[idt_v], add=True)
    plsc.subcore_barrier()

    @pl.when(s == 0)
    def _():
        pltpu.sync_copy(acc, deg_out.at[c])


@functools.partial(
    pl.kernel,
    out_type=jax.ShapeDtypeStruct((NC, NPAD, D), jnp.float32),
    mesh=_mesh,
    scratch_types=[
        pltpu.VMEM((G, K), jnp.int32),
        pltpu.VMEM((G, K), jnp.int32),
        pltpu.VMEM((K, D), jnp.float32),
        pltpu.VMEM_SHARED((NPAD, D), jnp.float32),
        pltpu.SemaphoreType.DMA,
    ],
)
def _aggr_kernel(y_hbm, src_hbm, dst_hbm, zeros_hbm, part_out,
                 src_v, dst_v, rows_v, acc, sem):
    c = lax.axis_index("c")
    s = lax.axis_index("s")
    t = c * NS + s
    pltpu.sync_copy(zeros_hbm.at[pl.ds(s * RPT, RPT)],
                    acc.at[pl.ds(s * RPT, RPT)])
    pltpu.sync_copy(src_hbm.at[t], src_v)
    pltpu.sync_copy(dst_hbm.at[t], dst_v)
    plsc.subcore_barrier()

    @pl.loop(0, G)
    def _(g):
        pltpu.async_copy(y_hbm.at[src_v.at[g]], rows_v, sem).wait()
        pltpu.sync_copy(rows_v, acc.at[dst_v.at[g]], add=True)

    plsc.subcore_barrier()
    pltpu.sync_copy(acc.at[pl.ds(s * RPT, RPT)],
                    part_out.at[c, pl.ds(s * RPT, RPT)])


def _dinv_diag(d_ref):
    deg = d_ref[0, 0, 0, :] + d_ref[1, 0, 0, :]
    dv = lax.rsqrt(jnp.maximum(deg, 1.0))
    rows = lax.broadcasted_iota(jnp.int32, (K, K), 0)
    cols = lax.broadcasted_iota(jnp.int32, (K, K), 1)
    return jnp.where(rows == cols, dv[None, :], 0.0)


def _scale_body(d_ref, x_ref, y_ref):
    dmat = _dinv_diag(d_ref)
    y_ref[...] = jnp.dot(dmat, x_ref[...], preferred_element_type=jnp.float32)


def _scale(deg4, x_pad):
    return pl.pallas_call(
        _scale_body,
        grid=(NB,),
        in_specs=[
            pl.BlockSpec((NC, 1, 1, K), lambda g: (0, g, 0, 0)),
            pl.BlockSpec((K, D), lambda g: (g, 0)),
        ],
        out_specs=pl.BlockSpec((K, D), lambda g: (g, 0)),
        out_shape=jax.ShapeDtypeStruct((NPAD, D), jnp.float32),
    )(deg4, x_pad)


def _update_body(d_ref, p0_ref, p1_ref, w_ref, b_ref, o_ref):
    dmat = _dinv_diag(d_ref)
    aggr = jnp.dot(dmat, p0_ref[...] + p1_ref[...],
                   preferred_element_type=jnp.float32)
    o_ref[...] = lax.dot_general(
        aggr, w_ref[...], (((1,), (1,)), ((), ())),
        preferred_element_type=jnp.float32) + b_ref[...]


def _update(deg4, p0, p1, W, b2):
    return pl.pallas_call(
        _update_body,
        grid=(NB,),
        in_specs=[
            pl.BlockSpec((NC, 1, 1, K), lambda g: (0, g, 0, 0)),
            pl.BlockSpec((K, D), lambda g: (g, 0)),
            pl.BlockSpec((K, D), lambda g: (g, 0)),
            pl.BlockSpec((D, D), lambda g: (0, 0)),
            pl.BlockSpec((1, D), lambda g: (0, 0)),
        ],
        out_specs=pl.BlockSpec((K, D), lambda g: (g, 0)),
        out_shape=jax.ShapeDtypeStruct((NPAD, D), jnp.float32),
    )(deg4, p0, p1, W, b2)


def kernel(node_embed, edges, W, b):
    src = edges[:, 0]
    dst = edges[:, 1]
    pad = E_PAD - E
    src_p = jnp.concatenate(
        [src, jnp.zeros((pad,), jnp.int32)]).reshape(NW, G, K)
    dst_p = jnp.concatenate(
        [dst, jnp.full((pad,), PAD_ROW, jnp.int32)]).reshape(NW, G, K)
    zeros_nd = jnp.zeros((NPAD, D), jnp.float32)
    iota_nb = jnp.arange(NB, dtype=jnp.int32)
    x_pad = jnp.concatenate(
        [node_embed, jnp.zeros((NPAD - N, D), jnp.float32)])

    zeros_nb = jnp.zeros((NB, K), jnp.float32)
    deg = _deg_kernel(dst_p, iota_nb, zeros_nb)
    deg4 = deg.reshape(NC, NB, 1, K)
    y = _scale(deg4, x_pad)
    parts = _aggr_kernel(y, src_p, dst_p, zeros_nd)
    out = _update(deg4, parts[0], parts[1], W, b.reshape(1, D))
    return out[:N]


# speedup vs baseline: 14.5822x; 1.0766x over previous
import functools

import jax
import jax.numpy as jnp
from jax import lax
from jax.experimental import pallas as pl
from jax.experimental.pallas import tpu as pltpu
from jax.experimental.pallas import tpu_sc as plsc

N = 10000
E = 320000
D = 128

NC = 2
NS = 16
NW = NC * NS
K = 128
G = -(-E // (NW * K))
E_PAD = NW * G * K
NB = 79
W128 = 128
NPAD = NB * W128
PAD_ROW = 10048
RPT = NPAD // NS

_mesh = plsc.VectorSubcoreMesh(core_axis_name="c", subcore_axis_name="s")


@functools.partial(
    pl.kernel,
    out_type=jax.ShapeDtypeStruct((NC, NB, W128), jnp.float32),
    mesh=_mesh,
    compiler_params=pltpu.CompilerParams(needs_layout_passes=False),
    scratch_types=[
        pltpu.VMEM((G, K), jnp.int32),
        pltpu.VMEM((NB, W128), jnp.float32),
        pltpu.VMEM((NB,), jnp.int32),
        pltpu.VMEM_SHARED((NB, W128), jnp.float32),
    ],
)
def _deg_kernel(dst_hbm, iota_hbm, zeros_hbm, deg_out, dst_v, deg_v, idt_v, acc):
    c = lax.axis_index("c")
    s = lax.axis_index("s")
    t = c * NS + s

    @pl.when(s == 0)
    def _():
        pltpu.sync_copy(zeros_hbm, acc)

    pltpu.sync_copy(dst_hbm.at[t], dst_v)
    pltpu.sync_copy(iota_hbm, idt_v)

    @pl.loop(0, NB)
    def _(r):
        @pl.loop(0, W128 // 16)
        def _(j):
            deg_v[r, pl.ds(j * 16, 16)] = jnp.zeros((16,), jnp.float32)

    ones = jnp.ones((16,), jnp.float32)

    @pl.loop(0, G)
    def _(g):
        @pl.loop(0, K // 16)
        def _(j):
            idx = dst_v[g, pl.ds(j * 16, 16)]
            plsc.addupdate_scatter(deg_v, [idx >> 7, idx & 127], ones)

    plsc.subcore_barrier()
    pltpu.sync_copy(deg_v, acc.at[idt_v], add=True)
    plsc.subcore_barrier()

    @pl.when(s == 0)
    def _():
        pltpu.sync_copy(acc, deg_out.at[c])


@functools.partial(
    pl.kernel,
    out_type=jax.ShapeDtypeStruct((NC, NPAD, D), jnp.float32),
    mesh=_mesh,
    scratch_types=[
        pltpu.VMEM((G, K), jnp.int32),
        pltpu.VMEM((2, 1, K), jnp.int32),
        pltpu.VMEM((2, K, D), jnp.float32),
        pltpu.VMEM_SHARED((NPAD, D), jnp.float32),
        pltpu.SemaphoreType.DMA((2,)),
        pltpu.SemaphoreType.DMA((2,)),
    ],
)
def _aggr_kernel(y_hbm, src_hbm, dst_hbm, zeros_hbm, part_out,
                 src_v, dstb, rows_v, acc, semg, semd):
    c = lax.axis_index("c")
    s = lax.axis_index("s")
    t = c * NS + s
    pltpu.sync_copy(zeros_hbm.at[pl.ds(s * RPT, RPT)],
                    acc.at[pl.ds(s * RPT, RPT)])
    pltpu.sync_copy(src_hbm.at[t], src_v)
    plsc.subcore_barrier()

    pltpu.async_copy(dst_hbm.at[t, 0], dstb.at[0], semd.at[0])
    pltpu.async_copy(dst_hbm.at[t, 1], dstb.at[1], semd.at[1])
    pltpu.async_copy(y_hbm.at[src_v.at[0]], rows_v.at[0], semg.at[0])

    @pl.loop(0, G)
    def _(g):
        b = g % 2

        @pl.when(g < G - 1)
        def _():
            pltpu.async_copy(y_hbm.at[src_v.at[g + 1]],
                             rows_v.at[1 - b], semg.at[1 - b])

        pltpu.make_async_copy(y_hbm.at[src_v.at[g]],
                              rows_v.at[b], semg.at[b]).wait()
        pltpu.make_async_copy(dst_hbm.at[t, g], dstb.at[b], semd.at[b]).wait()
        pltpu.sync_copy(rows_v.at[b], acc.at[dstb.at[b, 0]], add=True)

        @pl.when(g < G - 2)
        def _():
            pltpu.async_copy(dst_hbm.at[t, g + 2], dstb.at[b], semd.at[b])

    plsc.subcore_barrier()
    pltpu.sync_copy(acc.at[pl.ds(s * RPT, RPT)],
                    part_out.at[c, pl.ds(s * RPT, RPT)])


def _dinv_diag(d_ref):
    deg = d_ref[0, 0, 0, :] + d_ref[1, 0, 0, :]
    dv = lax.rsqrt(jnp.maximum(deg, 1.0))
    rows = lax.broadcasted_iota(jnp.int32, (W128, W128), 0)
    cols = lax.broadcasted_iota(jnp.int32, (W128, W128), 1)
    return jnp.where(rows == cols, dv[None, :], 0.0)


def _scale_body(d_ref, x_ref, y_ref):
    dmat = _dinv_diag(d_ref)
    y_ref[...] = jnp.dot(dmat, x_ref[...], preferred_element_type=jnp.float32)


def _scale(deg4, x_pad):
    return pl.pallas_call(
        _scale_body,
        grid=(NB,),
        in_specs=[
            pl.BlockSpec((NC, 1, 1, W128), lambda g: (0, g, 0, 0)),
            pl.BlockSpec((W128, D), lambda g: (g, 0)),
        ],
        out_specs=pl.BlockSpec((W128, D), lambda g: (g, 0)),
        out_shape=jax.ShapeDtypeStruct((NPAD, D), jnp.float32),
    )(deg4, x_pad)


def _update_body(d_ref, p0_ref, p1_ref, w_ref, b_ref, o_ref):
    dmat = _dinv_diag(d_ref)
    aggr = jnp.dot(dmat, p0_ref[...] + p1_ref[...],
                   preferred_element_type=jnp.float32)
    o_ref[...] = lax.dot_general(
        aggr, w_ref[...], (((1,), (1,)), ((), ())),
        preferred_element_type=jnp.float32) + b_ref[...]


def _update(deg4, p0, p1, W, b2):
    return pl.pallas_call(
        _update_body,
        grid=(NB,),
        in_specs=[
            pl.BlockSpec((NC, 1, 1, W128), lambda g: (0, g, 0, 0)),
            pl.BlockSpec((W128, D), lambda g: (g, 0)),
            pl.BlockSpec((W128, D), lambda g: (g, 0)),
            pl.BlockSpec((D, D), lambda g: (0, 0)),
            pl.BlockSpec((1, D), lambda g: (0, 0)),
        ],
        out_specs=pl.BlockSpec((W128, D), lambda g: (g, 0)),
        out_shape=jax.ShapeDtypeStruct((NPAD, D), jnp.float32),
    )(deg4, p0, p1, W, b2)


def kernel(node_embed, edges, W, b):
    src = edges[:, 0]
    dst = edges[:, 1]
    pad = E_PAD - E
    src_p = jnp.concatenate(
        [src, jnp.zeros((pad,), jnp.int32)]).reshape(NW, G, K)
    dst_p = jnp.concatenate(
        [dst, jnp.full((pad,), PAD_ROW, jnp.int32)]).reshape(NW, G, K)
    zeros_nd = jnp.zeros((NPAD, D), jnp.float32)
    iota_nb = jnp.arange(NB, dtype=jnp.int32)
    x_pad = jnp.concatenate(
        [node_embed, jnp.zeros((NPAD - N, D), jnp.float32)])

    zeros_nb = jnp.zeros((NB, W128), jnp.float32)
    deg = _deg_kernel(dst_p, iota_nb, zeros_nb)
    deg4 = deg.reshape(NC, NB, 1, W128)
    y = _scale(deg4, x_pad)
    dst_p4 = dst_p.reshape(NW, G, 1, K)
    parts = _aggr_kernel(y, src_p, dst_p4, zeros_nd)
    out = _update(deg4, parts[0], parts[1], W, b.reshape(1, D))
    return out[:N]


# speedup vs baseline: 21.3674x; 1.4653x over previous
import functools

import jax
import jax.numpy as jnp
from jax import lax
from jax.experimental import pallas as pl
from jax.experimental.pallas import tpu as pltpu
from jax.experimental.pallas import tpu_sc as plsc

N = 10000
E = 320000
D = 128

NC = 2
NS = 16
NW = NC * NS
K = 128
G = -(-E // (NW * K))
E_PAD = NW * G * K
NB = 79
W128 = 128
NPAD = NB * W128
PAD_ROW = 10048
RPT = NPAD // NS

_mesh = plsc.VectorSubcoreMesh(core_axis_name="c", subcore_axis_name="s")


@functools.partial(
    pl.kernel,
    out_type=jax.ShapeDtypeStruct((NC, NB, W128), jnp.float32),
    mesh=_mesh,
    compiler_params=pltpu.CompilerParams(needs_layout_passes=False),
    scratch_types=[
        pltpu.VMEM((G, K), jnp.int32),
        pltpu.VMEM((NB, W128), jnp.float32),
        pltpu.VMEM((NB,), jnp.int32),
        pltpu.VMEM_SHARED((NB, W128), jnp.float32),
    ],
)
def _deg_kernel(dst_hbm, iota_hbm, zeros_hbm, deg_out, dst_v, deg_v, idt_v, acc):
    c = lax.axis_index("c")
    s = lax.axis_index("s")
    t = c * NS + s

    @pl.when(s == 0)
    def _():
        pltpu.sync_copy(zeros_hbm, acc)

    pltpu.sync_copy(dst_hbm.at[t], dst_v)
    pltpu.sync_copy(iota_hbm, idt_v)

    @pl.loop(0, NB)
    def _(r):
        @pl.loop(0, W128 // 16)
        def _(j):
            deg_v[r, pl.ds(j * 16, 16)] = jnp.zeros((16,), jnp.float32)

    ones = jnp.ones((16,), jnp.float32)

    @pl.loop(0, G)
    def _(g):
        @pl.loop(0, K // 16)
        def _(j):
            idx = dst_v[g, pl.ds(j * 16, 16)]
            plsc.addupdate_scatter(deg_v, [idx >> 7, idx & 127], ones)

    plsc.subcore_barrier()
    pltpu.sync_copy(deg_v, acc.at[idt_v], add=True)
    plsc.subcore_barrier()

    @pl.when(s == 0)
    def _():
        pltpu.sync_copy(acc, deg_out.at[c])


@functools.partial(
    pl.kernel,
    out_type=jax.ShapeDtypeStruct((NC, NPAD, D), jnp.float32),
    mesh=_mesh,
    scratch_types=[
        pltpu.VMEM((G, K), jnp.int32),
        pltpu.VMEM((2, 1, K), jnp.int32),
        pltpu.VMEM((2, K, D), jnp.float32),
        pltpu.VMEM_SHARED((NPAD, D), jnp.float32),
        pltpu.SemaphoreType.DMA((2,)),
        pltpu.SemaphoreType.DMA((2,)),
    ],
)
def _aggr_kernel(y_hbm, src_hbm, dst_hbm, zeros_hbm, part_out,
                 src_v, dstb, rows_v, acc, semg, semd):
    c = lax.axis_index("c")
    s = lax.axis_index("s")
    t = c * NS + s
    pltpu.sync_copy(zeros_hbm.at[pl.ds(s * RPT, RPT)],
                    acc.at[pl.ds(s * RPT, RPT)])
    pltpu.sync_copy(src_hbm.at[t], src_v)
    plsc.subcore_barrier()

    y_c = y_hbm.at[c]
    pltpu.async_copy(dst_hbm.at[t, 0], dstb.at[0], semd.at[0])
    pltpu.async_copy(dst_hbm.at[t, 1], dstb.at[1], semd.at[1])
    pltpu.async_copy(y_c.at[src_v.at[0]], rows_v.at[0], semg.at[0])

    @pl.loop(0, G)
    def _(g):
        b = g % 2

        @pl.when(g < G - 1)
        def _():
            pltpu.async_copy(y_c.at[src_v.at[g + 1]],
                             rows_v.at[1 - b], semg.at[1 - b])

        pltpu.make_async_copy(y_c.at[src_v.at[g]],
                              rows_v.at[b], semg.at[b]).wait()
        pltpu.make_async_copy(dst_hbm.at[t, g], dstb.at[b], semd.at[b]).wait()
        pltpu.sync_copy(rows_v.at[b], acc.at[dstb.at[b, 0]], add=True)

        @pl.when(g < G - 2)
        def _():
            pltpu.async_copy(dst_hbm.at[t, g + 2], dstb.at[b], semd.at[b])

    plsc.subcore_barrier()
    pltpu.sync_copy(acc.at[pl.ds(s * RPT, RPT)],
                    part_out.at[c, pl.ds(s * RPT, RPT)])


def _dinv_col(d_ref):
    deg = d_ref[0] + d_ref[1]
    dv = lax.rsqrt(jnp.maximum(deg, 1.0))
    cols = [jnp.transpose(dv[g:g + 1, :]) for g in range(NB)]
    return jnp.concatenate(cols, axis=0)


def _scale_body(d_ref, x_ref, y_ref):
    y = x_ref[...] * _dinv_col(d_ref)
    y_ref[0] = y
    y_ref[1] = y


def _scale(deg, x_pad):
    return pl.pallas_call(
        _scale_body,
        out_shape=jax.ShapeDtypeStruct((NC, NPAD, D), jnp.float32),
    )(deg, x_pad)


def _update_body(d_ref, p0_ref, p1_ref, w_ref, b_ref, o_ref):
    aggr = (p0_ref[...] + p1_ref[...]) * _dinv_col(d_ref)
    o_ref[...] = lax.dot_general(
        aggr, w_ref[...], (((1,), (1,)), ((), ())),
        preferred_element_type=jnp.float32) + b_ref[...]


def _update(deg, p0, p1, W, b2):
    return pl.pallas_call(
        _update_body,
        out_shape=jax.ShapeDtypeStruct((NPAD, D), jnp.float32),
    )(deg, p0, p1, W, b2)


def kernel(node_embed, edges, W, b):
    src = edges[:, 0]
    dst = edges[:, 1]
    pad = E_PAD - E
    src_p = jnp.concatenate(
        [src, jnp.zeros((pad,), jnp.int32)]).reshape(NW, G, K)
    dst_p = jnp.concatenate(
        [dst, jnp.full((pad,), PAD_ROW, jnp.int32)]).reshape(NW, G, K)
    zeros_nd = jnp.zeros((NPAD, D), jnp.float32)
    iota_nb = jnp.arange(NB, dtype=jnp.int32)
    x_pad = jnp.concatenate(
        [node_embed, jnp.zeros((NPAD - N, D), jnp.float32)])

    zeros_nb = jnp.zeros((NB, W128), jnp.float32)
    deg = _deg_kernel(dst_p, iota_nb, zeros_nb)
    y = _scale(deg, x_pad)
    dst_p4 = dst_p.reshape(NW, G, 1, K)
    parts = _aggr_kernel(y, src_p, dst_p4, zeros_nd)
    out = _update(deg, parts[0], parts[1], W, b.reshape(1, D))
    return out[:N]
